# baseline (speedup 1.0000x reference)
import functools

import jax
import jax.numpy as jnp
from jax import lax
from jax.experimental import pallas as pl
from jax.experimental.pallas import tpu as pltpu
from jax.experimental.pallas import tpu_sc as plsc

N = 10000
D = 128
E = 320000
SCALE = 1.0
EPS = 1e-6

NC = 2
NS = 16
NW = NC * NS
C = 32
K = 320
NB = 8
EPAD = NW * K * C
H = 8
KH = K // H
NPAD = 10240
NPT = NPAD // NS

_MESH = plsc.VectorSubcoreMesh(core_axis_name="c", subcore_axis_name="s",
                               num_cores=NC, num_subcores=NS)


@functools.partial(
    pl.kernel, mesh=_MESH,
    out_type=jax.ShapeDtypeStruct((NC, NPAD, D), jnp.float32),
    scratch_types=[
        pltpu.VMEM((KH, C), jnp.int32),
        pltpu.VMEM((KH, C), jnp.int32),
        [pltpu.VMEM((C, D), jnp.float32) for _ in range(NB)],
        pltpu.VMEM_SHARED((NPAD, D), jnp.float32),
        [pltpu.SemaphoreType.DMA for _ in range(NB)],
    ])
def _sc_agg(feat, src_h, dst_h, z_h, agg_o,
            src_v, dst_v, ring, acc_s, sems):
    c = lax.axis_index("c")
    s = lax.axis_index("s")
    wid = c * NS + s

    pltpu.sync_copy(z_h, acc_s.at[pl.ds(s * NPT, NPT)])
    plsc.subcore_barrier()

    bufs = tuple(zip(ring, sems))
    for h in range(H):
        pltpu.sync_copy(src_h.at[wid, pl.ds(h * KH, KH)], src_v)
        pltpu.sync_copy(dst_h.at[wid, pl.ds(h * KH, KH)], dst_v)

        for p, (buf, sem) in enumerate(bufs):
            pltpu.async_copy(feat.at[src_v.at[p]], buf, sem)

        @pl.loop(0, KH - NB, step=NB)
        def _steady(k):
            for p, (buf, sem) in enumerate(bufs):
                pltpu.make_async_copy(feat.at[src_v.at[k + p]], buf, sem).wait()
                pltpu.sync_copy(buf, acc_s.at[dst_v.at[k + p]], add=True)
                pltpu.async_copy(feat.at[src_v.at[k + NB + p]], buf, sem)

        for p, (buf, sem) in enumerate(bufs):
            pltpu.make_async_copy(feat.at[src_v.at[KH - NB + p]], buf, sem).wait()
            pltpu.sync_copy(buf, acc_s.at[dst_v.at[KH - NB + p]], add=True)

    plsc.subcore_barrier()
    lo = s * NPT
    pltpu.sync_copy(acc_s.at[pl.ds(lo, NPT)], agg_o.at[c, pl.ds(lo, NPT)])


@functools.partial(
    pl.kernel, mesh=_MESH,
    out_type=jax.ShapeDtypeStruct((NC, NPAD, D), jnp.float32),
    scratch_types=[
        pltpu.VMEM((K, C), jnp.int32),
        pltpu.VMEM((C, D), jnp.float32),
        pltpu.VMEM_SHARED((NPAD, D), jnp.float32),
    ])
def _sc_deg(dst_h, zd_h, ones_h, deg_o, dst_v, ones_v, dacc_s):
    c = lax.axis_index("c")
    s = lax.axis_index("s")
    wid = c * NS + s

    pltpu.sync_copy(dst_h.at[wid], dst_v)
    pltpu.sync_copy(ones_h, ones_v)
    pltpu.sync_copy(zd_h, dacc_s.at[pl.ds(s * NPT, NPT)])
    plsc.subcore_barrier()

    @pl.loop(0, K)
    def _step(k):
        pltpu.sync_copy(ones_v, dacc_s.at[dst_v.at[k]], add=True)

    plsc.subcore_barrier()
    lo = s * NPT
    pltpu.sync_copy(dacc_s.at[pl.ds(lo, NPT)], deg_o.at[c, pl.ds(lo, NPT)])


def _mean_rows(aggp_ref, degp_ref):
    deg = degp_ref[0, :N] + degp_ref[1, :N]
    invd = 1.0 / jnp.maximum(deg, 1.0)
    mdeg = jnp.minimum(deg, 1.0)
    mean = (aggp_ref[0, :N] + aggp_ref[1, :N]) * invd
    return mean, mdeg


def _pairnorm(h):
    hc = h - jnp.mean(h, axis=0, keepdims=True)
    rms = jnp.sqrt(jnp.mean(jnp.sum(hc * hc, axis=1, keepdims=True)) + EPS)
    return hc * (SCALE / rms)


def _tc_layer1(aggp_ref, degp_ref, w_ref, b_ref, out_ref):
    mean, mdeg = _mean_rows(aggp_ref, degp_ref)
    h = jnp.dot(mean, w_ref[...], preferred_element_type=jnp.float32,
                precision=lax.Precision.HIGHEST)
    h = jax.nn.relu(h + b_ref[...] * mdeg)
    out_ref[...] = _pairnorm(h)


def _tc_layer2(aggp_ref, degp_ref, w_ref, b_ref, x0_ref, out_ref):
    mean, mdeg = _mean_rows(aggp_ref, degp_ref)
    h = jnp.dot(mean, w_ref[...], preferred_element_type=jnp.float32,
                precision=lax.Precision.HIGHEST)
    h = h + b_ref[...] * mdeg
    out_ref[...] = jax.nn.relu(x0_ref[...] + _pairnorm(h))


def kernel(x, edge_index, W1, b1, W2, b2):
    assert x.shape == (N, D) and edge_index.shape == (2, E)
    pad = EPAD - E
    pad_src = jnp.arange(pad, dtype=jnp.int32) % N
    src = jnp.concatenate([edge_index[0], pad_src]).reshape(NW, K, C)
    dst = jnp.concatenate(
        [edge_index[1], jnp.full((pad,), N, jnp.int32)]).reshape(NW, K, C)
    z = jnp.zeros((NPT, D), jnp.float32)
    ones = jnp.ones((C, D), jnp.float32)
    b1r = b1.reshape(1, D)
    b2r = b2.reshape(1, D)

    degp = _sc_deg(dst, z, ones)
    x_dep, _ = lax.optimization_barrier((x, degp))
    aggp = _sc_agg(x_dep, src, dst, z)
    degc = degp[:, :, :1]
    p1 = pl.pallas_call(
        _tc_layer1,
        out_shape=jax.ShapeDtypeStruct((N, D), jnp.float32),
    )(aggp, degc, W1, b1r)
    agg2p = _sc_agg(p1, src, dst, z)
    out = pl.pallas_call(
        _tc_layer2,
        out_shape=jax.ShapeDtypeStruct((N, D), jnp.float32),
    )(agg2p, degc, W2, b2r, x)
    return out


# speedup vs baseline: 1.1264x; 1.1264x over previous
import functools

import jax
import jax.numpy as jnp
from jax import lax
from jax.experimental import pallas as pl
from jax.experimental.pallas import tpu as pltpu
from jax.experimental.pallas import tpu_sc as plsc

N = 10000
D = 128
E = 320000
SCALE = 1.0
EPS = 1e-6

NC = 2
NS = 16
NW = NC * NS
C = 64
K = 160
NB = 4
EPAD = NW * K * C
H = 4
KH = K // H
NPAD = 10240
NPT = NPAD // NS

_MESH = plsc.VectorSubcoreMesh(core_axis_name="c", subcore_axis_name="s",
                               num_cores=NC, num_subcores=NS)


@functools.partial(
    pl.kernel, mesh=_MESH,
    out_type=jax.ShapeDtypeStruct((NC, NPAD, D), jnp.float32),
    scratch_types=[
        pltpu.VMEM((KH, C), jnp.int32),
        pltpu.VMEM((KH, C), jnp.int32),
        [pltpu.VMEM((C, D), jnp.float32) for _ in range(NB)],
        pltpu.VMEM_SHARED((NPAD, D), jnp.float32),
        [pltpu.SemaphoreType.DMA for _ in range(NB)],
    ])
def _sc_agg(feat, src_h, dst_h, z_h, agg_o,
            src_v, dst_v, ring, acc_s, sems):
    c = lax.axis_index("c")
    s = lax.axis_index("s")
    wid = c * NS + s

    pltpu.sync_copy(z_h, acc_s.at[pl.ds(s * NPT, NPT)])
    plsc.subcore_barrier()

    bufs = tuple(zip(ring, sems))
    for h in range(H):
        pltpu.sync_copy(src_h.at[wid, pl.ds(h * KH, KH)], src_v)
        pltpu.sync_copy(dst_h.at[wid, pl.ds(h * KH, KH)], dst_v)

        for p, (buf, sem) in enumerate(bufs):
            pltpu.async_copy(feat.at[src_v.at[p]], buf, sem)

        @pl.loop(0, KH - NB, step=NB)
        def _steady(k):
            for p, (buf, sem) in enumerate(bufs):
                pltpu.make_async_copy(feat.at[src_v.at[k + p]], buf, sem).wait()
                pltpu.sync_copy(buf, acc_s.at[dst_v.at[k + p]], add=True)
                pltpu.async_copy(feat.at[src_v.at[k + NB + p]], buf, sem)

        for p, (buf, sem) in enumerate(bufs):
            pltpu.make_async_copy(feat.at[src_v.at[KH - NB + p]], buf, sem).wait()
            pltpu.sync_copy(buf, acc_s.at[dst_v.at[KH - NB + p]], add=True)

    plsc.subcore_barrier()
    lo = s * NPT
    pltpu.sync_copy(acc_s.at[pl.ds(lo, NPT)], agg_o.at[c, pl.ds(lo, NPT)])


@functools.partial(
    pl.kernel, mesh=_MESH,
    out_type=jax.ShapeDtypeStruct((NC, NPAD, D), jnp.float32),
    scratch_types=[
        pltpu.VMEM((K, C), jnp.int32),
        pltpu.VMEM((C, D), jnp.float32),
        pltpu.VMEM_SHARED((NPAD, D), jnp.float32),
    ])
def _sc_deg(dst_h, zd_h, ones_h, deg_o, dst_v, ones_v, dacc_s):
    c = lax.axis_index("c")
    s = lax.axis_index("s")
    wid = c * NS + s

    pltpu.sync_copy(dst_h.at[wid], dst_v)
    pltpu.sync_copy(ones_h, ones_v)
    pltpu.sync_copy(zd_h, dacc_s.at[pl.ds(s * NPT, NPT)])
    plsc.subcore_barrier()

    @pl.loop(0, K)
    def _step(k):
        pltpu.sync_copy(ones_v, dacc_s.at[dst_v.at[k]], add=True)

    plsc.subcore_barrier()
    lo = s * NPT
    pltpu.sync_copy(dacc_s.at[pl.ds(lo, NPT)], deg_o.at[c, pl.ds(lo, NPT)])


def _mean_rows(aggp_ref, degp_ref):
    deg = degp_ref[0, :N] + degp_ref[1, :N]
    invd = 1.0 / jnp.maximum(deg, 1.0)
    mdeg = jnp.minimum(deg, 1.0)
    mean = (aggp_ref[0, :N] + aggp_ref[1, :N]) * invd
    return mean, mdeg


def _pairnorm(h):
    hc = h - jnp.mean(h, axis=0, keepdims=True)
    rms = jnp.sqrt(jnp.mean(jnp.sum(hc * hc, axis=1, keepdims=True)) + EPS)
    return hc * (SCALE / rms)


def _tc_layer1(aggp_ref, degp_ref, w_ref, b_ref, out_ref):
    mean, mdeg = _mean_rows(aggp_ref, degp_ref)
    h = jnp.dot(mean, w_ref[...], preferred_element_type=jnp.float32,
                precision=lax.Precision.HIGHEST)
    h = jax.nn.relu(h + b_ref[...] * mdeg)
    out_ref[...] = _pairnorm(h)


def _tc_layer2(aggp_ref, degp_ref, w_ref, b_ref, x0_ref, out_ref):
    mean, mdeg = _mean_rows(aggp_ref, degp_ref)
    h = jnp.dot(mean, w_ref[...], preferred_element_type=jnp.float32,
                precision=lax.Precision.HIGHEST)
    h = h + b_ref[...] * mdeg
    out_ref[...] = jax.nn.relu(x0_ref[...] + _pairnorm(h))


def kernel(x, edge_index, W1, b1, W2, b2):
    assert x.shape == (N, D) and edge_index.shape == (2, E)
    pad = EPAD - E
    pad_src = jnp.arange(pad, dtype=jnp.int32) % N
    src = jnp.concatenate([edge_index[0], pad_src]).reshape(NW, K, C)
    dst = jnp.concatenate(
        [edge_index[1], jnp.full((pad,), N, jnp.int32)]).reshape(NW, K, C)
    z = jnp.zeros((NPT, D), jnp.float32)
    ones = jnp.ones((C, D), jnp.float32)
    b1r = b1.reshape(1, D)
    b2r = b2.reshape(1, D)

    degp = _sc_deg(dst, z, ones)
    x_dep, _ = lax.optimization_barrier((x, degp))
    aggp = _sc_agg(x_dep, src, dst, z)
    degc = degp[:, :, :1]
    p1 = pl.pallas_call(
        _tc_layer1,
        out_shape=jax.ShapeDtypeStruct((N, D), jnp.float32),
    )(aggp, degc, W1, b1r)
    agg2p = _sc_agg(p1, src, dst, z)
    out = pl.pallas_call(
        _tc_layer2,
        out_shape=jax.ShapeDtypeStruct((N, D), jnp.float32),
    )(agg2p, degc, W2, b2r, x)
    return out
